# baseline (speedup 1.0000x reference)
import jax
import jax.numpy as jnp
from jax.experimental import pallas as pl
from jax.experimental.pallas import tpu as pltpu

_NB = 8


def _gcn_body(adj_ref, x_ref, w1_ref, b1_ref, w2_ref, b2_ref, alpha_ref,
              out_ref, at_s, deg_s, h0_s):
    k = pl.program_id(0)
    n = at_s.shape[0]
    rb = adj_ref.shape[0]

    @pl.when(k < _NB)
    def _stream():
        blk = adj_ref[...]
        rloc = jax.lax.broadcasted_iota(jnp.int32, blk.shape, 0)
        cols = jax.lax.broadcasted_iota(jnp.int32, blk.shape, 1)
        ab = jnp.where(cols == rloc + k * rb, jnp.float32(1.0),
                       (blk != 0).astype(jnp.float32)).astype(jnp.bfloat16)
        at_blk = ab.T
        at_s[:, pl.ds(k * rb, rb)] = at_blk
        dsum = jnp.sum(at_blk.astype(jnp.float32), axis=1, keepdims=True)
        deg_s[...] = jnp.where(k == 0, 0.0, deg_s[...]) + dsum

    @pl.when(k == 0)
    def _dense1():
        h0_s[...] = jnp.dot(x_ref[...].astype(jnp.bfloat16),
                            w1_ref[...].astype(jnp.bfloat16),
                            preferred_element_type=jnp.float32)

    @pl.when(k == _NB)
    def _compute():
        s = jax.lax.rsqrt(deg_s[...])
        at = at_s[...]
        y1 = (s * h0_s[...]).astype(jnp.bfloat16)
        c1 = jnp.dot(at, y1, preferred_element_type=jnp.float32)
        h1 = jax.nn.relu(s * c1 + b1_ref[...])
        g = jnp.dot(h1.astype(jnp.bfloat16), w2_ref[...].astype(jnp.bfloat16),
                    preferred_element_type=jnp.float32)
        y2 = (s * g).astype(jnp.bfloat16)
        c2 = jnp.dot(at, y2, preferred_element_type=jnp.float32)
        out_ref[...] = x_ref[...] + alpha_ref[0, 0] * (s * c2 + b2_ref[...])


def kernel(x, adj_matrix, W1, b1, W2, b2, alpha):
    n, in_dim = x.shape
    hid = W1.shape[1]
    rb = n // _NB
    const = lambda shape: pl.BlockSpec(shape, lambda k: (0, 0))
    call = pl.pallas_call(
        _gcn_body,
        grid=(_NB + 1,),
        in_specs=[
            pl.BlockSpec((rb, n), lambda k: (jnp.minimum(k, _NB - 1), 0)),
            const((n, in_dim)),
            const((in_dim, hid)),
            const((1, hid)),
            const((hid, in_dim)),
            const((1, in_dim)),
            const((1, 1)),
        ],
        out_specs=const((n, in_dim)),
        out_shape=jax.ShapeDtypeStruct((n, in_dim), jnp.float32),
        scratch_shapes=[
            pltpu.VMEM((n, n), jnp.bfloat16),
            pltpu.VMEM((n, 1), jnp.float32),
            pltpu.VMEM((n, hid), jnp.float32),
        ],
        compiler_params=pltpu.CompilerParams(
            vmem_limit_bytes=100 * 1024 * 1024,
        ),
    )
    return call(adj_matrix, x, W1, b1.reshape(1, hid), W2,
                b2.reshape(1, in_dim), jnp.asarray(alpha).reshape(1, 1))
